# baseline (speedup 1.0000x reference)
import functools

import jax
import jax.numpy as jnp
from jax import lax
from jax.experimental import pallas as pl
from jax.experimental.pallas import tpu as pltpu
from jax.experimental.pallas import tpu_sc as plsc

N = 10000
E = 320000
NFEAT = 128
NHID = 256
NCLASS = 64
NLAYERS = 8

NSUB = 16
HALF = NCLASS // 2
CW = 128
SUP = 4
E2 = 327680
RPT = E2 // NSUB // CW
NSUP = RPT // SUP
NSUP2 = NSUP // 2
NP = 10240
ROWS_PT = NP // NSUB
ZR = ROWS_PT // 4
BM = 1000


def _dense_in_body(x_ref, w_in_ref, b_in_ref, w_out_ref, out_ref):
    h = jnp.dot(x_ref[...], w_in_ref[...], preferred_element_type=jnp.float32)
    h = jnp.maximum(h + b_in_ref[...], 0.0)
    out_ref[...] = jnp.dot(h, w_out_ref[...], preferred_element_type=jnp.float32)


def _dense_in(x, w_in, b_in, w_out):
    return pl.pallas_call(
        _dense_in_body,
        grid=(N // BM,),
        in_specs=[
            pl.BlockSpec((BM, NFEAT), lambda i: (i, 0)),
            pl.BlockSpec((NFEAT, NHID), lambda i: (0, 0)),
            pl.BlockSpec((1, NHID), lambda i: (0, 0)),
            pl.BlockSpec((NHID, NCLASS), lambda i: (0, 0)),
        ],
        out_specs=pl.BlockSpec((BM, NCLASS), lambda i: (i, 0)),
        out_shape=jax.ShapeDtypeStruct((N, NCLASS), jnp.float32),
    )(x, w_in, b_in, w_out)


def _softmax_body(g_ref, b_ref, out_ref):
    y = g_ref[...] + b_ref[...]
    m = jnp.max(y, axis=1, keepdims=True)
    z = y - m
    lse = jnp.log(jnp.sum(jnp.exp(z), axis=1, keepdims=True))
    out_ref[...] = z - lse


def _softmax(g, b_out):
    return pl.pallas_call(
        _softmax_body,
        grid=(N // BM,),
        in_specs=[
            pl.BlockSpec((BM, NCLASS), lambda i: (i, 0)),
            pl.BlockSpec((1, NCLASS), lambda i: (0, 0)),
        ],
        out_specs=pl.BlockSpec((BM, NCLASS), lambda i: (i, 0)),
        out_shape=jax.ShapeDtypeStruct((N, NCLASS), jnp.float32),
    )(g, b_out)


def _prop(g0, src1, dst2, w, a1p, a2p):
    mesh = plsc.VectorSubcoreMesh(core_axis_name="c", subcore_axis_name="s")

    @functools.partial(
        pl.kernel,
        mesh=mesh,
        compiler_params=pltpu.CompilerParams(
            needs_layout_passes=False, use_tc_tiling_on_sc=False),
        out_type=[
            jax.ShapeDtypeStruct((2 * NP, HALF), jnp.float32),
        ],
        scratch_types=[
            pltpu.VMEM_SHARED((NP, HALF), jnp.float32),
            pltpu.VMEM_SHARED((NP, HALF), jnp.float32),
            pltpu.VMEM((ROWS_PT, HALF), jnp.float32),
            pltpu.VMEM((ZR, HALF), jnp.float32),
            pltpu.VMEM((6, SUP, CW), jnp.int32),
            pltpu.VMEM((6, SUP, CW), jnp.int32),
            pltpu.VMEM((6, SUP * CW), jnp.float32),
            pltpu.VMEM((SUP * CW, HALF), jnp.float32),
            pltpu.VMEM((SUP * CW, HALF), jnp.float32),
            pltpu.VMEM((SUP * CW, HALF), jnp.float32),
            pltpu.VMEM((16, 16), jnp.float32),
            pltpu.VMEM((16, 16), jnp.float32),
            pltpu.SemaphoreType.DMA,
            pltpu.SemaphoreType.DMA,
            pltpu.SemaphoreType.DMA,
            pltpu.SemaphoreType.DMA,
            pltpu.SemaphoreType.DMA,
            pltpu.SemaphoreType.DMA,
            pltpu.SemaphoreType.DMA,
            pltpu.SemaphoreType.DMA,
            pltpu.SemaphoreType.DMA,
            pltpu.SemaphoreType.DMA,
            pltpu.SemaphoreType.DMA,
            pltpu.SemaphoreType.DMA,
        ],
    )
    def prop_kernel(g0_hbm, src1_hbm, dst2_hbm, w_hbm, a1_hbm, a2_hbm,
                    out_q, gA_sh, gB_sh, g0_v, zero_v,
                    srcR, dstR, wR, rows0, rows1, rows2,
                    a1_v, a2_v, gs0, gs1, gs2, ss0, ss1, ss2,
                    is0, is1, is2, is3, is4, is5):
        c = lax.axis_index("c")
        s = lax.axis_index("s")
        row0 = s * ROWS_PT
        gbase = c * NP + row0
        rb_loc = s * RPT

        ROWS = (rows0, rows1, rows2)
        GS = (gs0, gs1, gs2)
        SS = (ss0, ss1, ss2)
        IS = (is0, is1, is2, is3, is4, is5)

        def idx_refs(r6):
            return (srcR.at[r6], dstR.at[r6], wR.at[r6])

        def load_idx(cc, r6):
            sr, dr, wr = idx_refs(r6)
            pltpu.async_copy(src1_hbm.at[pl.ds(rb_loc + cc * SUP, SUP)], sr, IS[r6])
            pltpu.async_copy(dst2_hbm.at[pl.ds(rb_loc + cc * SUP, SUP)], dr, IS[r6])
            pltpu.async_copy(w_hbm.at[pl.ds((rb_loc + cc * SUP) * CW, SUP * CW)],
                             wr, IS[r6])

        def wait_idx(cc, r6):
            sr, dr, wr = idx_refs(r6)
            pltpu.make_async_copy(src1_hbm.at[pl.ds(rb_loc + cc * SUP, SUP)],
                                  sr, IS[r6]).wait()
            pltpu.make_async_copy(dst2_hbm.at[pl.ds(rb_loc + cc * SUP, SUP)],
                                  dr, IS[r6]).wait()
            pltpu.make_async_copy(w_hbm.at[pl.ds((rb_loc + cc * SUP) * CW, SUP * CW)],
                                  wr, IS[r6]).wait()

        def gather(gin, r, r6):
            for j in range(SUP):
                pltpu.async_copy(gin.at[srcR.at[r6].at[j]],
                                 ROWS[r].at[pl.ds(j * CW, CW)], GS[r])

        def wait_gather(gin, r, r6):
            for j in range(SUP):
                pltpu.make_async_copy(gin.at[srcR.at[r6].at[j]],
                                      ROWS[r].at[pl.ds(j * CW, CW)], GS[r]).wait()

        def scatter(gacc, r, r6):
            for j in range(SUP):
                pltpu.async_copy(ROWS[r].at[pl.ds(j * CW, CW)],
                                 gacc.at[dstR.at[r6].at[j]], SS[r], add=True)

        def wait_scatter(gacc, r, r6):
            for j in range(SUP):
                pltpu.make_async_copy(ROWS[r].at[pl.ds(j * CW, CW)],
                                      gacc.at[dstR.at[r6].at[j]], SS[r]).wait()

        def multiply(r, r6):
            rowsx, wx = ROWS[r], wR.at[r6]

            def body(k, carry):
                for u in range(8):
                    e = k * 8 + u
                    wb = plsc.load_gather(wx, [jnp.full((16,), 0, jnp.int32) + e])
                    rowsx[e, pl.ds(0, 16)] = rowsx[e, pl.ds(0, 16)] * wb
                    rowsx[e, pl.ds(16, 16)] = rowsx[e, pl.ds(16, 16)] * wb
                return carry

            lax.fori_loop(0, SUP * CW // 8, body, 0)

        pltpu.sync_copy(a1_hbm, a1_v)
        pltpu.sync_copy(a2_hbm, a2_v)
        pltpu.sync_copy(g0_hbm.at[pl.ds(gbase, ROWS_PT)], g0_v)
        pltpu.sync_copy(g0_v, gA_sh.at[pl.ds(row0, ROWS_PT)])

        def zero_body(i, carry):
            zero_v[i, pl.ds(0, 16)] = jnp.zeros((16,), jnp.float32)
            zero_v[i, pl.ds(16, 16)] = jnp.zeros((16,), jnp.float32)
            return carry

        lax.fori_loop(0, ZR, zero_body, 0)

        def zero_slice(dst_sh):
            for z in range(ROWS_PT // ZR):
                pltpu.sync_copy(zero_v, dst_sh.at[pl.ds(row0 + z * ZR, ZR)])

        zero_slice(gB_sh)
        plsc.subcore_barrier()

        def do_layer(l, gin, gacc):
            load_idx(0, 0)
            load_idx(1, 1)
            wait_idx(0, 0)
            gather(gin, 0, 0)
            wait_idx(1, 1)
            gather(gin, 1, 1)
            load_idx(2, 2)
            load_idx(3, 3)

            def process(ct, off):
                r = off % 3
                rn = (off + 2) % 3
                q = off % 6
                q2 = (off + 2) % 6
                q4 = (off + 4) % 6
                q5 = (off + 5) % 6

                @pl.when(ct < NSUP)
                def _():
                    wait_gather(gin, r, q)
                    multiply(r, q)

                    @pl.when(ct >= 1)
                    def _():
                        wait_scatter(gacc, rn, q5)

                    @pl.when(ct + 2 < NSUP)
                    def _():
                        wait_idx(ct + 2, q2)
                        gather(gin, rn, q2)

                    @pl.when(ct + 4 < NSUP)
                    def _():
                        load_idx(ct + 4, q4)

                    scatter(gacc, r, q)

            def iter_body(k6, carry):
                for off in range(6):
                    process(k6 * 6 + off, off)
                return carry

            lax.fori_loop(0, (NSUP + 5) // 6, iter_body, 0)
            wait_scatter(gacc, (NSUP - 1) % 3, (NSUP - 1) % 6)
            plsc.subcore_barrier()

            a1b = a1_v[l, pl.ds(0, 16)]
            a2b = a2_v[l, pl.ds(0, 16)]

            def comb_pass(buf, base, nrows):
                pltpu.sync_copy(gacc.at[pl.ds(row0 + base, nrows)],
                                buf.at[pl.ds(0, nrows)])

                def comb_body(i, carry):
                    for j in (0, 16):
                        v = buf[i, pl.ds(j, 16)] * a1b \
                            + g0_v[base + i, pl.ds(j, 16)] * a2b
                        buf[i, pl.ds(j, 16)] = v
                    return carry

                lax.fori_loop(0, nrows, comb_body, 0)
                pltpu.sync_copy(buf.at[pl.ds(0, nrows)],
                                gacc.at[pl.ds(row0 + base, nrows)])

            comb_pass(rows0, 0, SUP * CW)
            comb_pass(rows1, SUP * CW, ROWS_PT - SUP * CW)
            zero_slice(gin)
            plsc.subcore_barrier()

        def pair_body(m, carry):
            do_layer(2 * m, gA_sh, gB_sh)
            do_layer(2 * m + 1, gB_sh, gA_sh)
            return carry

        lax.fori_loop(0, NLAYERS // 2, pair_body, 0)

        for base, buf, nrows in ((0, rows0, SUP * CW),
                                 (SUP * CW, rows1, ROWS_PT - SUP * CW)):
            pltpu.sync_copy(gA_sh.at[pl.ds(row0 + base, nrows)],
                            buf.at[pl.ds(0, nrows)])
            pltpu.sync_copy(buf.at[pl.ds(0, nrows)],
                            out_q.at[pl.ds(gbase + base, nrows)])

    return prop_kernel(g0, src1, dst2, w, a1p, a2p)


def kernel(x, edge_index, edge_weight, W_in, b_in, W_out, b_out, alpha1, alpha2):
    g0 = _dense_in(x, W_in, b_in.reshape(1, NHID), W_out)
    g0_pad = jnp.pad(g0, ((0, NP - N), (0, 0)))
    g0_split = g0_pad.reshape(NP, 2, HALF).transpose(1, 0, 2).reshape(2 * NP, HALF)

    src = edge_index[1].astype(jnp.int32)
    dst = edge_index[0].astype(jnp.int32)
    src_p = jnp.pad(src, (0, E2 - E))
    dst_p = jnp.pad(dst, (0, E2 - E), constant_values=N)
    w_p = jnp.pad(edge_weight, (0, E2 - E))
    src1 = src_p.reshape(E2 // CW, CW)
    dst2 = dst_p.reshape(E2 // CW, CW)
    a1p = jnp.tile(jnp.pad(alpha1, (0, 16 - NLAYERS)).reshape(16, 1), (1, 16))
    a2p = jnp.tile(jnp.pad(alpha2, (0, 16 - NLAYERS)).reshape(16, 1), (1, 16))

    q, = _prop(g0_split, src1, dst2, w_p, a1p, a2p)
    g = q.reshape(2, NP, HALF)[:, :N].transpose(1, 0, 2).reshape(N, NCLASS)
    return _softmax(g, b_out.reshape(1, NCLASS))


# speedup vs baseline: 1.4251x; 1.4251x over previous
import functools

import jax
import jax.numpy as jnp
from jax import lax
from jax.experimental import pallas as pl
from jax.experimental.pallas import tpu as pltpu
from jax.experimental.pallas import tpu_sc as plsc

N = 10000
E = 320000
NFEAT = 128
NHID = 256
NCLASS = 64
NLAYERS = 8

NSUB = 16
HALF = NCLASS // 2
CW = 128
SUP = 4
E2 = 327680
RPT = E2 // NSUB // CW
NSUP = RPT // SUP
NSUP2 = NSUP // 2
NP = 10240
ROWS_PT = NP // NSUB
ZR = ROWS_PT // 4
BM = 1000


def _dense_in_body(x_ref, w_in_ref, b_in_ref, w_out_ref, out_ref):
    h = jnp.dot(x_ref[...], w_in_ref[...], preferred_element_type=jnp.float32)
    h = jnp.maximum(h + b_in_ref[...], 0.0)
    out_ref[...] = jnp.dot(h, w_out_ref[...], preferred_element_type=jnp.float32)


def _dense_in(x, w_in, b_in, w_out):
    return pl.pallas_call(
        _dense_in_body,
        grid=(N // BM,),
        in_specs=[
            pl.BlockSpec((BM, NFEAT), lambda i: (i, 0)),
            pl.BlockSpec((NFEAT, NHID), lambda i: (0, 0)),
            pl.BlockSpec((1, NHID), lambda i: (0, 0)),
            pl.BlockSpec((NHID, NCLASS), lambda i: (0, 0)),
        ],
        out_specs=pl.BlockSpec((BM, NCLASS), lambda i: (i, 0)),
        out_shape=jax.ShapeDtypeStruct((N, NCLASS), jnp.float32),
    )(x, w_in, b_in, w_out)


def _softmax_body(g_ref, b_ref, out_ref):
    y = g_ref[...] + b_ref[...]
    m = jnp.max(y, axis=1, keepdims=True)
    z = y - m
    lse = jnp.log(jnp.sum(jnp.exp(z), axis=1, keepdims=True))
    out_ref[...] = z - lse


def _softmax(g, b_out):
    return pl.pallas_call(
        _softmax_body,
        grid=(N // BM,),
        in_specs=[
            pl.BlockSpec((BM, NCLASS), lambda i: (i, 0)),
            pl.BlockSpec((1, NCLASS), lambda i: (0, 0)),
        ],
        out_specs=pl.BlockSpec((BM, NCLASS), lambda i: (i, 0)),
        out_shape=jax.ShapeDtypeStruct((N, NCLASS), jnp.float32),
    )(g, b_out)


def _prop(g0, src1, dst2, w, a1p, a2p):
    mesh = plsc.VectorSubcoreMesh(core_axis_name="c", subcore_axis_name="s")

    @functools.partial(
        pl.kernel,
        mesh=mesh,
        compiler_params=pltpu.CompilerParams(
            needs_layout_passes=False, use_tc_tiling_on_sc=False),
        out_type=[
            jax.ShapeDtypeStruct((2 * NP, HALF), jnp.float32),
        ],
        scratch_types=[
            pltpu.VMEM_SHARED((NP, HALF), jnp.float32),
            pltpu.VMEM_SHARED((NP, HALF), jnp.float32),
            pltpu.VMEM((ROWS_PT, HALF), jnp.float32),
            pltpu.VMEM((ZR, HALF), jnp.float32),
            pltpu.VMEM((6, SUP, CW), jnp.int32),
            pltpu.VMEM((6, SUP, CW), jnp.int32),
            pltpu.VMEM((6, SUP * CW), jnp.float32),
            pltpu.VMEM((SUP * CW, HALF), jnp.float32),
            pltpu.VMEM((SUP * CW, HALF), jnp.float32),
            pltpu.VMEM((SUP * CW, HALF), jnp.float32),
            pltpu.VMEM((16, 16), jnp.float32),
            pltpu.VMEM((16, 16), jnp.float32),
            pltpu.SemaphoreType.DMA,
            pltpu.SemaphoreType.DMA,
            pltpu.SemaphoreType.DMA,
            pltpu.SemaphoreType.DMA,
            pltpu.SemaphoreType.DMA,
            pltpu.SemaphoreType.DMA,
            pltpu.SemaphoreType.DMA,
            pltpu.SemaphoreType.DMA,
            pltpu.SemaphoreType.DMA,
            pltpu.SemaphoreType.DMA,
            pltpu.SemaphoreType.DMA,
            pltpu.SemaphoreType.DMA,
        ],
    )
    def prop_kernel(g0_hbm, src1_hbm, dst2_hbm, w_hbm, a1_hbm, a2_hbm,
                    out_q, gA_sh, gB_sh, g0_v, zero_v,
                    srcR, dstR, wR, rows0, rows1, rows2,
                    a1_v, a2_v, gs0, gs1, gs2, ss0, ss1, ss2,
                    is0, is1, is2, is3, is4, is5):
        c = lax.axis_index("c")
        s = lax.axis_index("s")
        row0 = s * ROWS_PT
        gbase = c * NP + row0
        rb_loc = s * RPT

        ROWS = (rows0, rows1, rows2)
        GS = (gs0, gs1, gs2)
        SS = (ss0, ss1, ss2)
        IS = (is0, is1, is2, is3, is4, is5)

        def idx_refs(r6):
            return (srcR.at[r6], dstR.at[r6], wR.at[r6])

        def load_idx(cc, r6):
            sr, dr, wr = idx_refs(r6)
            pltpu.async_copy(src1_hbm.at[pl.ds(rb_loc + cc * SUP, SUP)], sr, IS[r6])
            pltpu.async_copy(dst2_hbm.at[pl.ds(rb_loc + cc * SUP, SUP)], dr, IS[r6])
            pltpu.async_copy(w_hbm.at[pl.ds((rb_loc + cc * SUP) * CW, SUP * CW)],
                             wr, IS[r6])

        def wait_idx(cc, r6):
            sr, dr, wr = idx_refs(r6)
            pltpu.make_async_copy(src1_hbm.at[pl.ds(rb_loc + cc * SUP, SUP)],
                                  sr, IS[r6]).wait()
            pltpu.make_async_copy(dst2_hbm.at[pl.ds(rb_loc + cc * SUP, SUP)],
                                  dr, IS[r6]).wait()
            pltpu.make_async_copy(w_hbm.at[pl.ds((rb_loc + cc * SUP) * CW, SUP * CW)],
                                  wr, IS[r6]).wait()

        def gather(gin, r, r6):
            for j in range(SUP):
                pltpu.async_copy(gin.at[srcR.at[r6].at[j]],
                                 ROWS[r].at[pl.ds(j * CW, CW)], GS[r])

        def wait_gather(gin, r, r6):
            for j in range(SUP):
                pltpu.make_async_copy(gin.at[srcR.at[r6].at[j]],
                                      ROWS[r].at[pl.ds(j * CW, CW)], GS[r]).wait()

        def scatter(gacc, r, r6):
            for j in range(SUP):
                pltpu.async_copy(ROWS[r].at[pl.ds(j * CW, CW)],
                                 gacc.at[dstR.at[r6].at[j]], SS[r], add=True)

        def wait_scatter(gacc, r, r6):
            for j in range(SUP):
                pltpu.make_async_copy(ROWS[r].at[pl.ds(j * CW, CW)],
                                      gacc.at[dstR.at[r6].at[j]], SS[r]).wait()

        def multiply(r, r6):
            rowsx, wx = ROWS[r], wR.at[r6]

            @plsc.parallel_loop(0, SUP * CW, 1, unroll=8)
            def _(e):
                wb = plsc.load_gather(wx, [jnp.full((16,), 0, jnp.int32) + e])
                rowsx[e, pl.ds(0, 16)] = rowsx[e, pl.ds(0, 16)] * wb
                rowsx[e, pl.ds(16, 16)] = rowsx[e, pl.ds(16, 16)] * wb

        pltpu.sync_copy(a1_hbm, a1_v)
        pltpu.sync_copy(a2_hbm, a2_v)
        pltpu.sync_copy(g0_hbm.at[pl.ds(gbase, ROWS_PT)], g0_v)
        pltpu.sync_copy(g0_v, gA_sh.at[pl.ds(row0, ROWS_PT)])

        def zero_body(i, carry):
            zero_v[i, pl.ds(0, 16)] = jnp.zeros((16,), jnp.float32)
            zero_v[i, pl.ds(16, 16)] = jnp.zeros((16,), jnp.float32)
            return carry

        lax.fori_loop(0, ZR, zero_body, 0)

        def zero_slice(dst_sh):
            for z in range(ROWS_PT // ZR):
                pltpu.sync_copy(zero_v, dst_sh.at[pl.ds(row0 + z * ZR, ZR)])

        zero_slice(gB_sh)
        plsc.subcore_barrier()

        def do_layer(l, gin, gacc):
            load_idx(0, 0)
            load_idx(1, 1)
            wait_idx(0, 0)
            gather(gin, 0, 0)
            wait_idx(1, 1)
            gather(gin, 1, 1)
            load_idx(2, 2)
            load_idx(3, 3)

            def process(ct, off):
                r = off % 3
                rn = (off + 2) % 3
                q = off % 6
                q2 = (off + 2) % 6
                q4 = (off + 4) % 6
                q5 = (off + 5) % 6

                @pl.when(ct < NSUP)
                def _():
                    wait_gather(gin, r, q)
                    multiply(r, q)

                    @pl.when(ct >= 1)
                    def _():
                        wait_scatter(gacc, rn, q5)

                    @pl.when(ct + 2 < NSUP)
                    def _():
                        wait_idx(ct + 2, q2)
                        gather(gin, rn, q2)

                    @pl.when(ct + 4 < NSUP)
                    def _():
                        load_idx(ct + 4, q4)

                    scatter(gacc, r, q)

            def iter_body(k6, carry):
                for off in range(6):
                    process(k6 * 6 + off, off)
                return carry

            lax.fori_loop(0, (NSUP + 5) // 6, iter_body, 0)
            wait_scatter(gacc, (NSUP - 1) % 3, (NSUP - 1) % 6)
            plsc.subcore_barrier()

            a1b = a1_v[l, pl.ds(0, 16)]
            a2b = a2_v[l, pl.ds(0, 16)]

            def comb_pass(buf, base, nrows):
                pltpu.sync_copy(gacc.at[pl.ds(row0 + base, nrows)],
                                buf.at[pl.ds(0, nrows)])

                def comb_body(i, carry):
                    for j in (0, 16):
                        v = buf[i, pl.ds(j, 16)] * a1b \
                            + g0_v[base + i, pl.ds(j, 16)] * a2b
                        buf[i, pl.ds(j, 16)] = v
                    return carry

                lax.fori_loop(0, nrows, comb_body, 0)
                pltpu.sync_copy(buf.at[pl.ds(0, nrows)],
                                gacc.at[pl.ds(row0 + base, nrows)])

            comb_pass(rows0, 0, SUP * CW)
            comb_pass(rows1, SUP * CW, ROWS_PT - SUP * CW)
            zero_slice(gin)
            plsc.subcore_barrier()

        def pair_body(m, carry):
            do_layer(2 * m, gA_sh, gB_sh)
            do_layer(2 * m + 1, gB_sh, gA_sh)
            return carry

        lax.fori_loop(0, NLAYERS // 2, pair_body, 0)

        for base, buf, nrows in ((0, rows0, SUP * CW),
                                 (SUP * CW, rows1, ROWS_PT - SUP * CW)):
            pltpu.sync_copy(gA_sh.at[pl.ds(row0 + base, nrows)],
                            buf.at[pl.ds(0, nrows)])
            pltpu.sync_copy(buf.at[pl.ds(0, nrows)],
                            out_q.at[pl.ds(gbase + base, nrows)])

    return prop_kernel(g0, src1, dst2, w, a1p, a2p)


def kernel(x, edge_index, edge_weight, W_in, b_in, W_out, b_out, alpha1, alpha2):
    g0 = _dense_in(x, W_in, b_in.reshape(1, NHID), W_out)
    g0_pad = jnp.pad(g0, ((0, NP - N), (0, 0)))
    g0_split = g0_pad.reshape(NP, 2, HALF).transpose(1, 0, 2).reshape(2 * NP, HALF)

    src = edge_index[1].astype(jnp.int32)
    dst = edge_index[0].astype(jnp.int32)
    src_p = jnp.pad(src, (0, E2 - E))
    dst_p = jnp.pad(dst, (0, E2 - E), constant_values=N)
    w_p = jnp.pad(edge_weight, (0, E2 - E))
    src1 = src_p.reshape(E2 // CW, CW)
    dst2 = dst_p.reshape(E2 // CW, CW)
    a1p = jnp.tile(jnp.pad(alpha1, (0, 16 - NLAYERS)).reshape(16, 1), (1, 16))
    a2p = jnp.tile(jnp.pad(alpha2, (0, 16 - NLAYERS)).reshape(16, 1), (1, 16))

    q, = _prop(g0_split, src1, dst2, w_p, a1p, a2p)
    g = q.reshape(2, NP, HALF)[:, :N].transpose(1, 0, 2).reshape(N, NCLASS)
    return _softmax(g, b_out.reshape(1, NCLASS))


# speedup vs baseline: 1.4541x; 1.0203x over previous
import functools

import jax
import jax.numpy as jnp
from jax import lax
from jax.experimental import pallas as pl
from jax.experimental.pallas import tpu as pltpu
from jax.experimental.pallas import tpu_sc as plsc

N = 10000
E = 320000
NFEAT = 128
NHID = 256
NCLASS = 64
NLAYERS = 8

NSUB = 16
HALF = NCLASS // 2
CW = 128
SUP = 4
E2 = 327680
RPT = E2 // NSUB // CW
NSUP = RPT // SUP
NSUP2 = NSUP // 2
NP = 10240
ROWS_PT = NP // NSUB
ZR = ROWS_PT // 4
BM = 1000


def _dense_in_body(x_ref, w_in_ref, b_in_ref, w_out_ref, out_ref):
    h = jnp.dot(x_ref[...], w_in_ref[...], preferred_element_type=jnp.float32)
    h = jnp.maximum(h + b_in_ref[...], 0.0)
    out_ref[...] = jnp.dot(h, w_out_ref[...], preferred_element_type=jnp.float32)


def _dense_in(x, w_in, b_in, w_out):
    return pl.pallas_call(
        _dense_in_body,
        grid=(N // BM,),
        in_specs=[
            pl.BlockSpec((BM, NFEAT), lambda i: (i, 0)),
            pl.BlockSpec((NFEAT, NHID), lambda i: (0, 0)),
            pl.BlockSpec((1, NHID), lambda i: (0, 0)),
            pl.BlockSpec((NHID, NCLASS), lambda i: (0, 0)),
        ],
        out_specs=pl.BlockSpec((BM, NCLASS), lambda i: (i, 0)),
        out_shape=jax.ShapeDtypeStruct((N, NCLASS), jnp.float32),
    )(x, w_in, b_in, w_out)


def _softmax_body(g_ref, b_ref, out_ref):
    y = g_ref[...] + b_ref[...]
    m = jnp.max(y, axis=1, keepdims=True)
    z = y - m
    lse = jnp.log(jnp.sum(jnp.exp(z), axis=1, keepdims=True))
    out_ref[...] = z - lse


def _softmax(g, b_out):
    return pl.pallas_call(
        _softmax_body,
        grid=(N // BM,),
        in_specs=[
            pl.BlockSpec((BM, NCLASS), lambda i: (i, 0)),
            pl.BlockSpec((1, NCLASS), lambda i: (0, 0)),
        ],
        out_specs=pl.BlockSpec((BM, NCLASS), lambda i: (i, 0)),
        out_shape=jax.ShapeDtypeStruct((N, NCLASS), jnp.float32),
    )(g, b_out)


def _prop(g0, src1, dst2, w, a1p, a2p):
    mesh = plsc.VectorSubcoreMesh(core_axis_name="c", subcore_axis_name="s")

    @functools.partial(
        pl.kernel,
        mesh=mesh,
        compiler_params=pltpu.CompilerParams(
            needs_layout_passes=False, use_tc_tiling_on_sc=False),
        out_type=[
            jax.ShapeDtypeStruct((2 * NP, HALF), jnp.float32),
        ],
        scratch_types=[
            pltpu.VMEM_SHARED((NP, HALF), jnp.float32),
            pltpu.VMEM_SHARED((NP, HALF), jnp.float32),
            pltpu.VMEM((ROWS_PT, HALF), jnp.float32),
            pltpu.VMEM((ZR, HALF), jnp.float32),
            pltpu.VMEM((6, SUP, CW), jnp.int32),
            pltpu.VMEM((6, SUP, CW), jnp.int32),
            pltpu.VMEM((6, SUP * CW), jnp.float32),
            pltpu.VMEM((SUP * CW, HALF), jnp.float32),
            pltpu.VMEM((SUP * CW, HALF), jnp.float32),
            pltpu.VMEM((SUP * CW, HALF), jnp.float32),
            pltpu.VMEM((16, 16), jnp.float32),
            pltpu.VMEM((16, 16), jnp.float32),
            pltpu.SemaphoreType.DMA,
            pltpu.SemaphoreType.DMA,
            pltpu.SemaphoreType.DMA,
            pltpu.SemaphoreType.DMA,
            pltpu.SemaphoreType.DMA,
            pltpu.SemaphoreType.DMA,
            pltpu.SemaphoreType.DMA,
            pltpu.SemaphoreType.DMA,
            pltpu.SemaphoreType.DMA,
            pltpu.SemaphoreType.DMA,
            pltpu.SemaphoreType.DMA,
            pltpu.SemaphoreType.DMA,
        ],
    )
    def prop_kernel(g0_hbm, src1_hbm, dst2_hbm, w_hbm, a1_hbm, a2_hbm,
                    out_q, gA_sh, gB_sh, g0_v, zero_v,
                    srcR, dstR, wR, rows0, rows1, rows2,
                    a1_v, a2_v, gs0, gs1, gs2, ss0, ss1, ss2,
                    is0, is1, is2, is3, is4, is5):
        c = lax.axis_index("c")
        s = lax.axis_index("s")
        row0 = s * ROWS_PT
        gbase = c * NP + row0
        rb_loc = s * RPT

        ROWS = (rows0, rows1, rows2)
        GS = (gs0, gs1, gs2)
        SS = (ss0, ss1, ss2)
        IS = (is0, is1, is2, is3, is4, is5)

        def idx_refs(r6):
            return (srcR.at[r6], dstR.at[r6], wR.at[r6])

        def load_idx(cc, r6):
            sr, dr, wr = idx_refs(r6)
            pltpu.async_copy(src1_hbm.at[pl.ds(rb_loc + cc * SUP, SUP)], sr, IS[r6])
            pltpu.async_copy(dst2_hbm.at[pl.ds(rb_loc + cc * SUP, SUP)], dr, IS[r6])
            pltpu.async_copy(w_hbm.at[pl.ds((rb_loc + cc * SUP) * CW, SUP * CW)],
                             wr, IS[r6])

        def wait_idx(cc, r6):
            sr, dr, wr = idx_refs(r6)
            pltpu.make_async_copy(src1_hbm.at[pl.ds(rb_loc + cc * SUP, SUP)],
                                  sr, IS[r6]).wait()
            pltpu.make_async_copy(dst2_hbm.at[pl.ds(rb_loc + cc * SUP, SUP)],
                                  dr, IS[r6]).wait()
            pltpu.make_async_copy(w_hbm.at[pl.ds((rb_loc + cc * SUP) * CW, SUP * CW)],
                                  wr, IS[r6]).wait()

        def gather(gin, r, r6):
            for j in range(SUP):
                pltpu.async_copy(gin.at[srcR.at[r6].at[j]],
                                 ROWS[r].at[pl.ds(j * CW, CW)], GS[r])

        def wait_gather(gin, r, r6):
            for j in range(SUP):
                pltpu.make_async_copy(gin.at[srcR.at[r6].at[j]],
                                      ROWS[r].at[pl.ds(j * CW, CW)], GS[r]).wait()

        def scatter(gacc, r, r6):
            for j in range(SUP):
                pltpu.async_copy(ROWS[r].at[pl.ds(j * CW, CW)],
                                 gacc.at[dstR.at[r6].at[j]], SS[r], add=True)

        def wait_scatter(gacc, r, r6):
            for j in range(SUP):
                pltpu.make_async_copy(ROWS[r].at[pl.ds(j * CW, CW)],
                                      gacc.at[dstR.at[r6].at[j]], SS[r]).wait()

        def multiply(r, r6):
            rowsx, wx = ROWS[r], wR.at[r6]

            @plsc.parallel_loop(0, SUP * CW, 1, unroll=8)
            def _(e):
                wb = plsc.load_gather(wx, [jnp.full((16,), 0, jnp.int32) + e])
                rowsx[e, pl.ds(0, 16)] = rowsx[e, pl.ds(0, 16)] * wb
                rowsx[e, pl.ds(16, 16)] = rowsx[e, pl.ds(16, 16)] * wb

        pltpu.sync_copy(a1_hbm, a1_v)
        pltpu.sync_copy(a2_hbm, a2_v)
        pltpu.sync_copy(g0_hbm.at[pl.ds(gbase, ROWS_PT)], g0_v)
        pltpu.sync_copy(g0_v, gA_sh.at[pl.ds(row0, ROWS_PT)])

        def zero_body(i, carry):
            zero_v[i, pl.ds(0, 16)] = jnp.zeros((16,), jnp.float32)
            zero_v[i, pl.ds(16, 16)] = jnp.zeros((16,), jnp.float32)
            return carry

        lax.fori_loop(0, ZR, zero_body, 0)

        def zero_slice(dst_sh):
            for z in range(ROWS_PT // ZR):
                pltpu.sync_copy(zero_v, dst_sh.at[pl.ds(row0 + z * ZR, ZR)])

        zero_slice(gB_sh)
        plsc.subcore_barrier()

        def do_layer(l, gin, gacc):
            load_idx(0, 0)
            load_idx(1, 1)
            wait_idx(0, 0)
            gather(gin, 0, 0)
            wait_idx(1, 1)
            gather(gin, 1, 1)
            load_idx(2, 2)
            load_idx(3, 3)

            def process(ct, off):
                r = off % 3
                rn = (off + 2) % 3
                q = off % 6
                q2 = (off + 2) % 6
                q4 = (off + 4) % 6
                q5 = (off + 5) % 6

                @pl.when(ct < NSUP)
                def _():
                    wait_gather(gin, r, q)
                    multiply(r, q)

                    @pl.when(ct >= 1)
                    def _():
                        wait_scatter(gacc, rn, q5)

                    @pl.when(ct + 2 < NSUP)
                    def _():
                        wait_idx(ct + 2, q2)
                        gather(gin, rn, q2)

                    @pl.when(ct + 4 < NSUP)
                    def _():
                        load_idx(ct + 4, q4)

                    scatter(gacc, r, q)

            def iter_body(k6, carry):
                for off in range(6):
                    process(k6 * 6 + off, off)
                return carry

            lax.fori_loop(0, (NSUP + 5) // 6, iter_body, 0)
            wait_scatter(gacc, (NSUP - 1) % 3, (NSUP - 1) % 6)
            plsc.subcore_barrier()

            a1b = a1_v[l, pl.ds(0, 16)]
            a2b = a2_v[l, pl.ds(0, 16)]

            def comb_pass(buf, base, nrows):
                pltpu.sync_copy(gacc.at[pl.ds(row0 + base, nrows)],
                                buf.at[pl.ds(0, nrows)])

                @plsc.parallel_loop(0, nrows, 1, unroll=8)
                def _(i):
                    for j in (0, 16):
                        v = buf[i, pl.ds(j, 16)] * a1b \
                            + g0_v[base + i, pl.ds(j, 16)] * a2b
                        buf[i, pl.ds(j, 16)] = v
                pltpu.sync_copy(buf.at[pl.ds(0, nrows)],
                                gacc.at[pl.ds(row0 + base, nrows)])

            comb_pass(rows0, 0, SUP * CW)
            comb_pass(rows1, SUP * CW, ROWS_PT - SUP * CW)
            zero_slice(gin)
            plsc.subcore_barrier()

        def pair_body(m, carry):
            do_layer(2 * m, gA_sh, gB_sh)
            do_layer(2 * m + 1, gB_sh, gA_sh)
            return carry

        lax.fori_loop(0, NLAYERS // 2, pair_body, 0)

        for base, buf, nrows in ((0, rows0, SUP * CW),
                                 (SUP * CW, rows1, ROWS_PT - SUP * CW)):
            pltpu.sync_copy(gA_sh.at[pl.ds(row0 + base, nrows)],
                            buf.at[pl.ds(0, nrows)])
            pltpu.sync_copy(buf.at[pl.ds(0, nrows)],
                            out_q.at[pl.ds(gbase + base, nrows)])

    return prop_kernel(g0, src1, dst2, w, a1p, a2p)


def kernel(x, edge_index, edge_weight, W_in, b_in, W_out, b_out, alpha1, alpha2):
    g0 = _dense_in(x, W_in, b_in.reshape(1, NHID), W_out)
    g0_pad = jnp.pad(g0, ((0, NP - N), (0, 0)))
    g0_split = g0_pad.reshape(NP, 2, HALF).transpose(1, 0, 2).reshape(2 * NP, HALF)

    src = edge_index[1].astype(jnp.int32)
    dst = edge_index[0].astype(jnp.int32)
    src_p = jnp.pad(src, (0, E2 - E))
    dst_p = jnp.pad(dst, (0, E2 - E), constant_values=N)
    w_p = jnp.pad(edge_weight, (0, E2 - E))
    src1 = src_p.reshape(E2 // CW, CW)
    dst2 = dst_p.reshape(E2 // CW, CW)
    a1p = jnp.tile(jnp.pad(alpha1, (0, 16 - NLAYERS)).reshape(16, 1), (1, 16))
    a2p = jnp.tile(jnp.pad(alpha2, (0, 16 - NLAYERS)).reshape(16, 1), (1, 16))

    q, = _prop(g0_split, src1, dst2, w_p, a1p, a2p)
    g = q.reshape(2, NP, HALF)[:, :N].transpose(1, 0, 2).reshape(N, NCLASS)
    return _softmax(g, b_out.reshape(1, NCLASS))


# speedup vs baseline: 1.5926x; 1.0952x over previous
import functools

import jax
import jax.numpy as jnp
from jax import lax
from jax.experimental import pallas as pl
from jax.experimental.pallas import tpu as pltpu
from jax.experimental.pallas import tpu_sc as plsc

N = 10000
E = 320000
NFEAT = 128
NHID = 256
NCLASS = 64
NLAYERS = 8

NSUB = 16
HALF = NCLASS // 2
CW = 128
SUP = 4
E2 = 327680
RPT = E2 // NSUB // CW
NSUP = RPT // SUP
NSUP2 = NSUP // 2
NP = 10240
ROWS_PT = NP // NSUB
ZR = ROWS_PT // 4
BM = 1000


def _dense_in_body(x_ref, w_in_ref, b_in_ref, w_out_ref, out_ref):
    h = jnp.dot(x_ref[...], w_in_ref[...], preferred_element_type=jnp.float32)
    h = jnp.maximum(h + b_in_ref[...], 0.0)
    out_ref[...] = jnp.dot(h, w_out_ref[...], preferred_element_type=jnp.float32)


def _dense_in(x, w_in, b_in, w_out):
    return pl.pallas_call(
        _dense_in_body,
        grid=(N // BM,),
        in_specs=[
            pl.BlockSpec((BM, NFEAT), lambda i: (i, 0)),
            pl.BlockSpec((NFEAT, NHID), lambda i: (0, 0)),
            pl.BlockSpec((1, NHID), lambda i: (0, 0)),
            pl.BlockSpec((NHID, NCLASS), lambda i: (0, 0)),
        ],
        out_specs=pl.BlockSpec((BM, NCLASS), lambda i: (i, 0)),
        out_shape=jax.ShapeDtypeStruct((N, NCLASS), jnp.float32),
    )(x, w_in, b_in, w_out)


def _softmax_body(g_ref, b_ref, out_ref):
    y = g_ref[...] + b_ref[...]
    m = jnp.max(y, axis=1, keepdims=True)
    z = y - m
    lse = jnp.log(jnp.sum(jnp.exp(z), axis=1, keepdims=True))
    out_ref[...] = z - lse


def _softmax(g, b_out):
    return pl.pallas_call(
        _softmax_body,
        grid=(N // BM,),
        in_specs=[
            pl.BlockSpec((BM, NCLASS), lambda i: (i, 0)),
            pl.BlockSpec((1, NCLASS), lambda i: (0, 0)),
        ],
        out_specs=pl.BlockSpec((BM, NCLASS), lambda i: (i, 0)),
        out_shape=jax.ShapeDtypeStruct((N, NCLASS), jnp.float32),
    )(g, b_out)


def _prop(g0, src1, dst2, w, a1p, a2p):
    mesh = plsc.VectorSubcoreMesh(core_axis_name="c", subcore_axis_name="s")

    @functools.partial(
        pl.kernel,
        mesh=mesh,
        compiler_params=pltpu.CompilerParams(
            needs_layout_passes=False, use_tc_tiling_on_sc=False),
        out_type=[
            jax.ShapeDtypeStruct((2 * NP, HALF), jnp.float32),
        ],
        scratch_types=[
            pltpu.VMEM_SHARED((NP, HALF), jnp.float32),
            pltpu.VMEM_SHARED((NP, HALF), jnp.float32),
            pltpu.VMEM((ROWS_PT, HALF), jnp.float32),
            pltpu.VMEM((ZR, HALF), jnp.float32),
            pltpu.VMEM((6, SUP, CW), jnp.int32),
            pltpu.VMEM((6, SUP, CW), jnp.int32),
            pltpu.VMEM((6, SUP * CW), jnp.float32),
            pltpu.VMEM((SUP * CW, HALF), jnp.float32),
            pltpu.VMEM((SUP * CW, HALF), jnp.float32),
            pltpu.VMEM((SUP * CW, HALF), jnp.float32),
            pltpu.VMEM((16, 16), jnp.float32),
            pltpu.VMEM((16, 16), jnp.float32),
            pltpu.SemaphoreType.DMA,
            pltpu.SemaphoreType.DMA,
            pltpu.SemaphoreType.DMA,
            pltpu.SemaphoreType.DMA,
            pltpu.SemaphoreType.DMA,
            pltpu.SemaphoreType.DMA,
            pltpu.SemaphoreType.DMA,
            pltpu.SemaphoreType.DMA,
            pltpu.SemaphoreType.DMA,
            pltpu.SemaphoreType.DMA,
            pltpu.SemaphoreType.DMA,
            pltpu.SemaphoreType.DMA,
        ],
    )
    def prop_kernel(g0_hbm, src1_hbm, dst2_hbm, w_hbm, a1_hbm, a2_hbm,
                    out_q, gA_sh, gB_sh, g0_v, zero_v,
                    srcR, dstR, wR, rows0, rows1, rows2,
                    a1_v, a2_v, gs0, gs1, gs2, ss0, ss1, ss2,
                    is0, is1, is2, is3, is4, is5):
        c = lax.axis_index("c")
        s = lax.axis_index("s")
        row0 = s * ROWS_PT
        gbase = c * NP + row0
        rb_loc = s * RPT

        ROWS = (rows0, rows1, rows2)
        GS = (gs0, gs1, gs2)
        SS = (ss0, ss1, ss2)
        IS = (is0, is1, is2, is3, is4, is5)

        def idx_refs(r6):
            return (srcR.at[r6], dstR.at[r6], wR.at[r6])

        def load_idx(cc, r6):
            sr, dr, wr = idx_refs(r6)
            pltpu.async_copy(src1_hbm.at[pl.ds(rb_loc + cc * SUP, SUP)], sr, IS[r6])
            pltpu.async_copy(dst2_hbm.at[pl.ds(rb_loc + cc * SUP, SUP)], dr, IS[r6])
            pltpu.async_copy(w_hbm.at[pl.ds((rb_loc + cc * SUP) * CW, SUP * CW)],
                             wr, IS[r6])

        def wait_idx(cc, r6):
            sr, dr, wr = idx_refs(r6)
            pltpu.make_async_copy(src1_hbm.at[pl.ds(rb_loc + cc * SUP, SUP)],
                                  sr, IS[r6]).wait()
            pltpu.make_async_copy(dst2_hbm.at[pl.ds(rb_loc + cc * SUP, SUP)],
                                  dr, IS[r6]).wait()
            pltpu.make_async_copy(w_hbm.at[pl.ds((rb_loc + cc * SUP) * CW, SUP * CW)],
                                  wr, IS[r6]).wait()

        def gather(gin, r, r6):
            for j in range(SUP):
                pltpu.async_copy(gin.at[srcR.at[r6].at[j]],
                                 ROWS[r].at[pl.ds(j * CW, CW)], GS[r])

        def wait_gather(gin, r, r6):
            for j in range(SUP):
                pltpu.make_async_copy(gin.at[srcR.at[r6].at[j]],
                                      ROWS[r].at[pl.ds(j * CW, CW)], GS[r]).wait()

        def scatter(gacc, r, r6):
            for j in range(SUP):
                pltpu.async_copy(ROWS[r].at[pl.ds(j * CW, CW)],
                                 gacc.at[dstR.at[r6].at[j]], SS[r], add=True)

        def wait_scatter(gacc, r, r6):
            for j in range(SUP):
                pltpu.make_async_copy(ROWS[r].at[pl.ds(j * CW, CW)],
                                      gacc.at[dstR.at[r6].at[j]], SS[r]).wait()

        def multiply(r, r6):
            rowsx, wx = ROWS[r], wR.at[r6]

            @plsc.parallel_loop(0, SUP * CW, 1, unroll=8)
            def _(e):
                wb = plsc.load_gather(wx, [jnp.full((16,), 0, jnp.int32) + e])
                rowsx[e, pl.ds(0, 16)] = rowsx[e, pl.ds(0, 16)] * wb
                rowsx[e, pl.ds(16, 16)] = rowsx[e, pl.ds(16, 16)] * wb

        pltpu.sync_copy(a1_hbm, a1_v)
        pltpu.sync_copy(a2_hbm, a2_v)
        pltpu.sync_copy(g0_hbm.at[pl.ds(gbase, ROWS_PT)], g0_v)
        pltpu.sync_copy(g0_v, gA_sh.at[pl.ds(row0, ROWS_PT)])

        def zero_body(i, carry):
            zero_v[i, pl.ds(0, 16)] = jnp.zeros((16,), jnp.float32)
            zero_v[i, pl.ds(16, 16)] = jnp.zeros((16,), jnp.float32)
            return carry

        lax.fori_loop(0, ZR, zero_body, 0)

        def zero_slice(dst_sh):
            for z in range(ROWS_PT // ZR):
                pltpu.sync_copy(zero_v, dst_sh.at[pl.ds(row0 + z * ZR, ZR)])

        zero_slice(gB_sh)
        plsc.subcore_barrier()

        def do_layer(l, gin, gacc):
            load_idx(0, 0)
            load_idx(1, 1)
            wait_idx(0, 0)
            gather(gin, 0, 0)
            wait_idx(1, 1)
            gather(gin, 1, 1)
            load_idx(2, 2)
            load_idx(3, 3)

            def process(ct, off):
                r = off % 3
                rn = (off + 2) % 3
                q = off % 6
                q2 = (off + 2) % 6
                q4 = (off + 4) % 6
                q5 = (off + 5) % 6

                @pl.when(ct < NSUP)
                def _():
                    wait_gather(gin, r, q)
                    multiply(r, q)
                    scatter(gacc, r, q)

                    @pl.when(ct >= 1)
                    def _():
                        wait_scatter(gacc, rn, q5)

                    @pl.when(ct + 2 < NSUP)
                    def _():
                        wait_idx(ct + 2, q2)
                        gather(gin, rn, q2)

                    @pl.when(ct + 4 < NSUP)
                    def _():
                        load_idx(ct + 4, q4)

            def iter_body(k6, carry):
                for off in range(6):
                    process(k6 * 6 + off, off)
                return carry

            lax.fori_loop(0, (NSUP + 5) // 6, iter_body, 0)
            wait_scatter(gacc, (NSUP - 1) % 3, (NSUP - 1) % 6)
            plsc.subcore_barrier()

            a1b = a1_v[l, pl.ds(0, 16)]
            a2b = a2_v[l, pl.ds(0, 16)]

            def comb_pass(buf, base, nrows):
                pltpu.sync_copy(gacc.at[pl.ds(row0 + base, nrows)],
                                buf.at[pl.ds(0, nrows)])

                @plsc.parallel_loop(0, nrows, 1, unroll=8)
                def _(i):
                    for j in (0, 16):
                        v = buf[i, pl.ds(j, 16)] * a1b \
                            + g0_v[base + i, pl.ds(j, 16)] * a2b
                        buf[i, pl.ds(j, 16)] = v
                pltpu.sync_copy(buf.at[pl.ds(0, nrows)],
                                gacc.at[pl.ds(row0 + base, nrows)])

            comb_pass(rows0, 0, SUP * CW)
            comb_pass(rows1, SUP * CW, ROWS_PT - SUP * CW)
            zero_slice(gin)
            plsc.subcore_barrier()

        def pair_body(m, carry):
            do_layer(2 * m, gA_sh, gB_sh)
            do_layer(2 * m + 1, gB_sh, gA_sh)
            return carry

        lax.fori_loop(0, NLAYERS // 2, pair_body, 0)

        for base, buf, nrows in ((0, rows0, SUP * CW),
                                 (SUP * CW, rows1, ROWS_PT - SUP * CW)):
            pltpu.sync_copy(gA_sh.at[pl.ds(row0 + base, nrows)],
                            buf.at[pl.ds(0, nrows)])
            pltpu.sync_copy(buf.at[pl.ds(0, nrows)],
                            out_q.at[pl.ds(gbase + base, nrows)])

    return prop_kernel(g0, src1, dst2, w, a1p, a2p)


def kernel(x, edge_index, edge_weight, W_in, b_in, W_out, b_out, alpha1, alpha2):
    g0 = _dense_in(x, W_in, b_in.reshape(1, NHID), W_out)
    g0_pad = jnp.pad(g0, ((0, NP - N), (0, 0)))
    g0_split = g0_pad.reshape(NP, 2, HALF).transpose(1, 0, 2).reshape(2 * NP, HALF)

    src = edge_index[1].astype(jnp.int32)
    dst = edge_index[0].astype(jnp.int32)
    src_p = jnp.pad(src, (0, E2 - E))
    dst_p = jnp.pad(dst, (0, E2 - E), constant_values=N)
    w_p = jnp.pad(edge_weight, (0, E2 - E))
    src1 = src_p.reshape(E2 // CW, CW)
    dst2 = dst_p.reshape(E2 // CW, CW)
    a1p = jnp.tile(jnp.pad(alpha1, (0, 16 - NLAYERS)).reshape(16, 1), (1, 16))
    a2p = jnp.tile(jnp.pad(alpha2, (0, 16 - NLAYERS)).reshape(16, 1), (1, 16))

    q, = _prop(g0_split, src1, dst2, w_p, a1p, a2p)
    g = q.reshape(2, NP, HALF)[:, :N].transpose(1, 0, 2).reshape(N, NCLASS)
    return _softmax(g, b_out.reshape(1, NCLASS))
